# baseline (speedup 1.0000x reference)
import functools

import jax
import jax.numpy as jnp
from jax import lax
from jax.experimental import pallas as pl
from jax.experimental.pallas import tpu as pltpu


def _packed_linear_body(pack, x_ref, w_ref, b_ref, o_ref):
    out_f, in_f = w_ref.shape
    w_blk = w_ref[...]
    b_blk = b_ref[...]
    if pack > 1:
        w_blk = pltpu.repeat(pltpu.repeat(w_blk, pack, axis=1), pack, axis=0)
        rows = lax.broadcasted_iota(jnp.int32, w_blk.shape, 0)
        cols = lax.broadcasted_iota(jnp.int32, w_blk.shape, 1)
        w_blk = jnp.where(rows // out_f == cols // in_f, w_blk, 0.0)
        b_blk = pltpu.repeat(b_blk, pack, axis=1)
    acc = lax.dot_general(
        x_ref[...], w_blk,
        dimension_numbers=(((1,), (1,)), ((), ())),
        preferred_element_type=jnp.float32,
    )
    o_ref[...] = (acc + b_blk).astype(o_ref.dtype)


def _packed_linear(x, weight, bias, pack, block_rows):
    B, IN = x.shape
    OUT = weight.shape[0]
    dense_rows = B // pack
    tile_rows = max(8, min(block_rows // pack, dense_rows))

    xd = x.reshape(dense_rows, pack * IN)
    yd = pl.pallas_call(
        functools.partial(_packed_linear_body, pack),
        out_shape=jax.ShapeDtypeStruct((dense_rows, pack * OUT), x.dtype),
        grid=(pl.cdiv(dense_rows, tile_rows),),
        in_specs=[
            pl.BlockSpec((tile_rows, pack * IN), lambda i: (i, 0)),
            pl.BlockSpec((OUT, IN), lambda i: (0, 0)),
            pl.BlockSpec((1, OUT), lambda i: (0, 0)),
        ],
        out_specs=pl.BlockSpec((tile_rows, pack * OUT), lambda i: (i, 0)),
        compiler_params=pltpu.CompilerParams(
            dimension_semantics=("parallel",),
        ),
    )(xd, weight, bias.reshape(1, OUT))
    return yd.reshape(B, OUT)


def kernel(x, weight, bias, block_rows=32768):
    B, IN = x.shape
    OUT, IN_w = weight.shape
    assert IN == IN_w
    pack = 128 // OUT if (128 % OUT == 0 and B % (128 // OUT) == 0) else 1
    return _packed_linear(x, weight, bias, pack, block_rows)


# speedup vs baseline: 1.1504x; 1.1504x over previous
import jax
import jax.numpy as jnp
from jax import lax
from jax.experimental import pallas as pl
from jax.experimental.pallas import tpu as pltpu


def _linear_body(x_ref, w_ref, b_ref, o_ref):
    acc = lax.dot_general(
        x_ref[...], w_ref[...],
        dimension_numbers=(((1,), (1,)), ((), ())),
        preferred_element_type=jnp.float32,
    )
    o_ref[...] = (acc + b_ref[...]).astype(o_ref.dtype)


def kernel(x, weight, bias, block_rows=8192):
    B, IN = x.shape
    OUT, IN_w = weight.shape
    assert IN == IN_w
    br = max(8, min(block_rows, B))
    return pl.pallas_call(
        _linear_body,
        out_shape=jax.ShapeDtypeStruct((B, OUT), x.dtype),
        grid=(pl.cdiv(B, br),),
        in_specs=[
            pl.BlockSpec((br, IN), lambda i: (i, 0)),
            pl.BlockSpec((OUT, IN), lambda i: (0, 0)),
            pl.BlockSpec((1, OUT), lambda i: (0, 0)),
        ],
        out_specs=pl.BlockSpec((br, OUT), lambda i: (i, 0)),
        compiler_params=pltpu.CompilerParams(
            dimension_semantics=("parallel",),
        ),
    )(x, weight, bias.reshape(1, OUT))


# speedup vs baseline: 19.3119x; 16.7866x over previous
import jax
import jax.numpy as jnp
from jax import lax
from jax.experimental import pallas as pl
from jax.experimental.pallas import tpu as pltpu


def _linear_t_body(xt_ref, w_ref, b_ref, o_ref):
    acc = lax.dot_general(
        w_ref[...], xt_ref[...],
        dimension_numbers=(((1,), (0,)), ((), ())),
        preferred_element_type=jnp.float32,
    )
    o_ref[...] = (acc + b_ref[...]).astype(o_ref.dtype)


def kernel(x, weight, bias, block_cols=65536):
    B, IN = x.shape
    OUT, IN_w = weight.shape
    assert IN == IN_w
    bn = max(128, min(block_cols, B))
    xt = x.T
    yt = pl.pallas_call(
        _linear_t_body,
        out_shape=jax.ShapeDtypeStruct((OUT, B), x.dtype),
        grid=(pl.cdiv(B, bn),),
        in_specs=[
            pl.BlockSpec((IN, bn), lambda i: (0, i)),
            pl.BlockSpec((OUT, IN), lambda i: (0, 0)),
            pl.BlockSpec((OUT, 1), lambda i: (0, 0)),
        ],
        out_specs=pl.BlockSpec((OUT, bn), lambda i: (0, i)),
        compiler_params=pltpu.CompilerParams(
            dimension_semantics=("parallel",),
        ),
    )(xt, weight, bias.reshape(OUT, 1))
    return yt.T
